# baseline (speedup 1.0000x reference)
import functools

import jax
import jax.numpy as jnp
from jax import lax
from jax.experimental import pallas as pl
from jax.experimental.pallas import tpu as pltpu
from jax.experimental.pallas import tpu_sc as plsc

N_NODES = 50000
N_EDGES = 800000
EMB = 32
RADIUS = 0.05

NC = 2
NS = 16
NW = NC * NS
CH = 128
CPW = 196
EPW = CH * CPW
EP = NW * EPW
NCH = EP // CH
EP4 = EP // 4
DUMMY0 = N_NODES
NPAD = 50176
RPS = NPAD // NS
ZR = 196
OUT_RPS = N_NODES // NS

NCH2 = NCH // 2
CPW2 = NCH2 // NW
EP8 = EP4 // 2

_MESH = dict(core_axis_name="c", subcore_axis_name="s",
             num_cores=NC, num_subcores=NS)


def _worker_id():
    return lax.axis_index("s") * NC + lax.axis_index("c")


def _pipe(n, ig, iw, wait_i, wait_o):
    groups = (n - 4) // 3
    ig(0, 0)
    ig(1, 1)
    wait_i(0)
    iw(0, 0)
    ig(2, 2)
    wait_i(1)
    iw(1, 1)

    def grp(k, carry):
        c = 3 * k + 3
        for p in range(3):
            wait_o(p)
            ig(c + p, p)
            wait_i((p + 2) % 3)
            iw(c + p - 1, (p + 2) % 3)
        return carry

    lax.fori_loop(0, groups, grp, 0)
    for c in range(3 + 3 * groups, n):
        wait_o(c % 3)
        ig(c, c % 3)
        wait_i((c - 1) % 3)
        iw(c - 1, (c - 1) % 3)
    wait_i((n - 1) % 3)
    iw(n - 1, (n - 1) % 3)
    for b in range(3):
        wait_o(b)


@functools.cache
def _make_gather2(half: int):
    def body(tbl, idxa, idxb, outa, outb, idx_m, rows3, g0, g1, g2,
             w0, w1, w2):
        base = _worker_id() * CPW2
        sg = [g0, g1, g2]
        sw = [w0, w1, w2]

        def wait_g(b):
            pltpu.make_async_copy(outa.at[0], rows3.at[b], sg[b]).wait()

        def wait_w(b):
            pltpu.make_async_copy(outa.at[0], rows3.at[b], sw[b]).wait()

        def stream(idx_hbm, out_hbm):
            pltpu.sync_copy(idx_hbm.at[pl.ds(half * NCH2 + base, CPW2)],
                            idx_m)

            def ig(c, b):
                pltpu.async_copy(tbl.at[idx_m.at[c]], rows3.at[b], sg[b])

            def iw(c, b):
                pltpu.async_copy(rows3.at[b], out_hbm.at[base + c], sw[b])

            _pipe(CPW2, ig, iw, wait_g, wait_w)

        stream(idxa, outa)
        stream(idxb, outb)

    return pl.kernel(
        body,
        out_type=(jax.ShapeDtypeStruct((NCH2, CH, EMB), jnp.float32),
                  jax.ShapeDtypeStruct((NCH2, CH, EMB), jnp.float32)),
        mesh=plsc.VectorSubcoreMesh(**_MESH),
        scratch_types=[
            pltpu.VMEM((CPW2, CH), jnp.int32),
            pltpu.VMEM((3, CH, EMB), jnp.float32),
            pltpu.SemaphoreType.DMA,
            pltpu.SemaphoreType.DMA,
            pltpu.SemaphoreType.DMA,
            pltpu.SemaphoreType.DMA,
            pltpu.SemaphoreType.DMA,
            pltpu.SemaphoreType.DMA,
        ],
        compiler_params=pltpu.CompilerParams(use_tc_tiling_on_sc=False),
    )


@functools.cache
def _make_scatter_add():
    def body(ma_hbm, mb_hbm, dst_hbm, out_hbm, idx_r, rows3, zero_v, acc_sh,
             l0, l1, l2, a0, a1, a2):
        c = lax.axis_index("c")
        s = lax.axis_index("s")
        base = _worker_id() * CPW2
        sl = [l0, l1, l2]
        sa = [a0, a1, a2]

        def wait_l(b):
            pltpu.make_async_copy(dst_hbm.at[0], idx_r.at[b], sl[b]).wait()
            pltpu.make_async_copy(ma_hbm.at[0], rows3.at[b], sl[b]).wait()

        def wait_a(b):
            pltpu.make_async_copy(ma_hbm.at[0], rows3.at[b], sa[b]).wait()

        def stream(m_hbm, half):
            def il(ck, b):
                pltpu.async_copy(dst_hbm.at[half * NCH2 + base + ck],
                                 idx_r.at[b], sl[b])
                pltpu.async_copy(m_hbm.at[base + ck], rows3.at[b], sl[b])

            def ia(ck, b):
                del ck
                pltpu.async_copy(rows3.at[b], acc_sh.at[idx_r.at[b]], sa[b],
                                 add=True)

            _pipe(CPW2, il, ia, wait_l, wait_a)

        zv = jnp.zeros((16,), jnp.float32)

        def zloop(i, carry):
            zero_v[i, pl.ds(0, 16)] = zv
            zero_v[i, pl.ds(16, 16)] = zv
            return carry

        lax.fori_loop(0, ZR, zloop, 0)

        def zcopy(j, carry):
            pltpu.sync_copy(zero_v, acc_sh.at[pl.ds(s * RPS + j * ZR, ZR)])
            return carry

        lax.fori_loop(0, RPS // ZR, zcopy, 0)
        plsc.subcore_barrier()

        stream(ma_hbm, 0)
        stream(mb_hbm, 1)
        plsc.subcore_barrier()

        pltpu.sync_copy(acc_sh.at[pl.ds(s * OUT_RPS, OUT_RPS)],
                        out_hbm.at[c, pl.ds(s * OUT_RPS, OUT_RPS)])

    return pl.kernel(
        body,
        out_type=jax.ShapeDtypeStruct((NC, N_NODES, EMB), jnp.float32),
        mesh=plsc.VectorSubcoreMesh(**_MESH),
        scratch_types=[
            pltpu.VMEM((3, CH), jnp.int32),
            pltpu.VMEM((3, CH, EMB), jnp.float32),
            pltpu.VMEM((ZR, EMB), jnp.float32),
            pltpu.VMEM_SHARED((NPAD, EMB), jnp.float32),
            pltpu.SemaphoreType.DMA,
            pltpu.SemaphoreType.DMA,
            pltpu.SemaphoreType.DMA,
            pltpu.SemaphoreType.DMA,
            pltpu.SemaphoreType.DMA,
            pltpu.SemaphoreType.DMA,
        ],
        compiler_params=pltpu.CompilerParams(use_tc_tiling_on_sc=False),
    )


BP = 2048
BN = 2000


_TC_PARAMS = pltpu.CompilerParams(dimension_semantics=("parallel",))


def _relu(v):
    return jnp.maximum(v, 0.0)


def _dot(a, b):
    return jnp.dot(a, b, preferred_element_type=jnp.float32)


def _full(shape):
    return pl.BlockSpec(shape, lambda i: (0, 0))


def _edge_init_body(xi_ref, xj_ref, w1_ref, b1_ref,
                    w2_ref, b2_ref, w3_ref, b3_ref, ef_ref):
    xi = xi_ref[...]
    xj = xj_ref[...]
    lane = lax.broadcasted_iota(jnp.int32, xi.shape, 1) % EMB
    dr = xi - xj
    sq = dr * dr
    r2 = sq + jnp.roll(sq, -1, axis=1)
    r = jnp.sqrt(r2) / RADIUS
    z = jnp.zeros_like(xi)
    feat = (jnp.where(lane < 2, dr / RADIUS, z)
            + jnp.where(lane == 2, jnp.roll(r, 2, axis=1), z)
            + jnp.where((lane >= 3) & (lane < 5), jnp.roll(xi, 1, axis=1), z)
            + jnp.where((lane >= 5) & (lane < 7), jnp.roll(xj, 3, axis=1), z)
            + jnp.where(lane == 7, jnp.roll(xi, 3, axis=1), z)
            + jnp.where(lane == 8, jnp.roll(xi, 4, axis=1), z)
            + jnp.where(lane == 9, jnp.roll(xi, 5, axis=1), z)
            + jnp.where(lane == 10, jnp.roll(xi, 6, axis=1), z))
    h = _relu(_dot(feat, w1_ref[...]) + b1_ref[0:1, :])
    h = _relu(_dot(h, w2_ref[...]) + b2_ref[0:1, :])
    ef_ref[...] = _dot(h, w3_ref[...]) + b3_ref[0:1, :]


def _edge_layer_body(ef_ref, xi_ref, xj_ref, w1a_ref, w1b_ref, w1c_ref,
                     b1_ref, w2_ref, b2_ref, w3_ref, b3_ref, m_ref, efo_ref):
    ef = ef_ref[...]
    h = _relu(_dot(ef, w1a_ref[...]) + _dot(xi_ref[...], w1b_ref[...])
              + _dot(xj_ref[...], w1c_ref[...]) + b1_ref[0:1, :])
    h = _relu(_dot(h, w2_ref[...]) + b2_ref[0:1, :])
    m = _dot(h, w3_ref[...]) + b3_ref[0:1, :]
    m_ref[...] = m
    efo_ref[...] = ef + m


def _edge_last_body(ef_ref, xi_ref, xj_ref, w1a_ref, w1b_ref, w1c_ref,
                    b1_ref, w2_ref, b2_ref, w3_ref, b3_ref, m_ref):
    h = _relu(_dot(ef_ref[...], w1a_ref[...]) + _dot(xi_ref[...], w1b_ref[...])
              + _dot(xj_ref[...], w1c_ref[...]) + b1_ref[0:1, :])
    h = _relu(_dot(h, w2_ref[...]) + b2_ref[0:1, :])
    m_ref[...] = _dot(h, w3_ref[...]) + b3_ref[0:1, :]


def _node_emb_body(nf_ref, w1_ref, b1_ref, w2_ref, b2_ref, w3_ref, b3_ref,
                   out_ref):
    h = _relu(_dot(nf_ref[...], w1_ref[...]) + b1_ref[0:1, :])
    h = _relu(_dot(h, w2_ref[...]) + b2_ref[0:1, :])
    out_ref[...] = _dot(h, w3_ref[...]) + b3_ref[0:1, :]


def _node_layer_body(nf_ref, p0_ref, p1_ref, w1a_ref, w1b_ref, b1_ref,
                     w2_ref, b2_ref, w3_ref, b3_ref, out_ref):
    nf = nf_ref[...]
    aggr = p0_ref[...] + p1_ref[...]
    h = _relu(_dot(nf, w1a_ref[...]) + _dot(aggr, w1b_ref[...]) + b1_ref[0:1, :])
    h = _relu(_dot(h, w2_ref[...]) + b2_ref[0:1, :])
    out_ref[...] = nf + _dot(h, w3_ref[...]) + b3_ref[0:1, :]


def _node_out_body(nf_ref, w1_ref, b1_ref, w2_ref, b2_ref, w3_ref, b3_ref,
                   out_ref):
    h = _relu(_dot(nf_ref[...], w1_ref[...]) + b1_ref[0:1, :])
    h = _relu(_dot(h, w2_ref[...]) + b2_ref[0:1, :])
    out_ref[...] = _dot(h, w3_ref[...]) + b3_ref[0:1, :]


def _row_spec(bs, w):
    return pl.BlockSpec((bs, w), lambda i: (i, 0))


def _prep_w(w):
    return jnp.asarray(w, jnp.float32).T


def _prep_b(b):
    return jnp.broadcast_to(jnp.asarray(b, jnp.float32)[None, :], (8, b.shape[0]))


def _bd4(w):
    return jax.scipy.linalg.block_diag(w, w, w, w)


def _prep_b4(b):
    return _prep_b(jnp.tile(jnp.asarray(b, jnp.float32), 4))


def kernel(x, edge_index, edge_attr, params):
    del edge_attr
    f32 = jnp.float32
    src = edge_index[0].astype(jnp.int32)
    dst = edge_index[1].astype(jnp.int32)

    nf8 = jnp.concatenate([x[:, 0:4], jnp.tile(x[:, 4:5], (1, 4))], axis=1)
    nf32 = jnp.pad(nf8.astype(f32), ((0, 0), (0, 24)))

    pad = EP - N_EDGES
    src_p = jnp.concatenate([src, jnp.zeros((pad,), jnp.int32)]).reshape(NCH, CH)
    dst_p = jnp.concatenate([dst, jnp.zeros((pad,), jnp.int32)]).reshape(NCH, CH)
    dummy = DUMMY0 + (jnp.arange(pad, dtype=jnp.int32) % (NPAD - N_NODES))
    dst_scat = jnp.concatenate([dst, dummy]).reshape(NCH, CH)

    prm = params

    (w1e, b1e), (w2e, b2e), (w3e, b3e) = prm['emb_edge']
    w1e_pad = jnp.pad(_prep_w(w1e), ((0, 21), (0, 0)))

    (w1n, b1n), (w2n, b2n), (w3n, b3n) = prm['emb_node']
    w1n_t = jnp.pad(_prep_w(w1n), ((0, 24), (0, 0)))

    (w1o, b1o), (w2o, b2o), (w3o, b3o) = prm['node_out']

    ge = EP8 // BP
    gn = N_NODES // BN

    ga = _make_gather2(0)
    gb = _make_gather2(1)

    def edge_init(xi_p, xj_p, wargs):
        return pl.pallas_call(
            _edge_init_body,
            grid=(ge,),
            in_specs=[_row_spec(BP, 128), _row_spec(BP, 128),
                      _full((128, 128)), _full((8, 128)), _full((128, 128)),
                      _full((8, 128)), _full((128, 128)), _full((8, 128))],
            out_specs=_row_spec(BP, 128),
            out_shape=jax.ShapeDtypeStruct((EP8, 128), f32),
            compiler_params=_TC_PARAMS,
        )(xi_p, xj_p, *wargs)

    def unpack(pair):
        return (jnp.reshape(pair[0], (EP8, 128)),
                jnp.reshape(pair[1], (EP8, 128)))

    xiA, xjA = unpack(ga(nf32, dst_p, src_p))
    xiB, xjB = unpack(gb(nf32, dst_p, src_p))

    ewargs = (_bd4(w1e_pad), _prep_b4(b1e), _bd4(_prep_w(w2e)),
              _prep_b4(b2e), _bd4(_prep_w(w3e)), _prep_b4(b3e))
    ef_A = edge_init(xiA, xjA, ewargs)
    ef_B = edge_init(xiB, xjB, ewargs)

    nf = pl.pallas_call(
        _node_emb_body,
        grid=(gn,),
        in_specs=[_row_spec(BN, 32),
                  _full((32, EMB)), _full((8, EMB)), _full((EMB, EMB)),
                  _full((8, EMB)), _full((EMB, EMB)), _full((8, EMB))],
        out_specs=_row_spec(BN, EMB),
        out_shape=jax.ShapeDtypeStruct((N_NODES, EMB), f32),
        compiler_params=_TC_PARAMS,
    )(nf32, w1n_t, _prep_b(b1n), _prep_w(w2n), _prep_b(b2n),
      _prep_w(w3n), _prep_b(b3n))

    n_layers = len(prm['layers'])
    for li, lp in enumerate(prm['layers']):
        (w1l, b1l), (w2l, b2l), (w3l, b3l) = lp['lin_edge']
        w1l_t = _prep_w(w1l)
        (w1m, b1m), (w2m, b2m), (w3m, b3m) = lp['lin_node']
        w1m_t = _prep_w(w1m)

        last = li == n_layers - 1
        wargs = (_bd4(w1l_t[0:EMB]), _bd4(w1l_t[EMB:2 * EMB]),
                 _bd4(w1l_t[2 * EMB:]), _prep_b4(b1l), _bd4(_prep_w(w2l)),
                 _prep_b4(b2l), _bd4(_prep_w(w3l)), _prep_b4(b3l))
        especs = [_row_spec(BP, 128), _row_spec(BP, 128), _row_spec(BP, 128),
                  _full((128, 384)), _full((128, 384)),
                  _full((128, 384)), _full((8, 384)),
                  _full((384, 384)), _full((8, 384)),
                  _full((384, 128)), _full((8, 128))]

        def edge_layer(ef_h, xi_p, xj_p):
            if last:
                m = pl.pallas_call(
                    _edge_last_body,
                    grid=(ge,),
                    in_specs=especs,
                    out_specs=_row_spec(BP, 128),
                    out_shape=jax.ShapeDtypeStruct((EP8, 128), f32),
                    compiler_params=_TC_PARAMS,
                )(ef_h, xi_p, xj_p, *wargs)
                return m, m
            return pl.pallas_call(
                _edge_layer_body,
                grid=(ge,),
                in_specs=especs,
                out_specs=(_row_spec(BP, 128), _row_spec(BP, 128)),
                out_shape=(jax.ShapeDtypeStruct((EP8, 128), f32),
                           jax.ShapeDtypeStruct((EP8, 128), f32)),
                compiler_params=_TC_PARAMS,
            )(ef_h, xi_p, xj_p, *wargs)

        xiA, xjA = unpack(ga(nf, dst_p, src_p))
        xiB, xjB = unpack(gb(nf, dst_p, src_p))
        m_A, ef_A = edge_layer(ef_A, xiA, xjA)
        m_B, ef_B = edge_layer(ef_B, xiB, xjB)

        parts = _make_scatter_add()(jnp.reshape(m_A, (NCH2, CH, EMB)),
                                    jnp.reshape(m_B, (NCH2, CH, EMB)),
                                    dst_scat)

        nf = pl.pallas_call(
            _node_layer_body,
            grid=(gn,),
            in_specs=[_row_spec(BN, EMB), _row_spec(BN, EMB),
                      _row_spec(BN, EMB),
                      _full((EMB, 2 * EMB)), _full((EMB, 2 * EMB)),
                      _full((8, 2 * EMB)), _full((2 * EMB, 2 * EMB)),
                      _full((8, 2 * EMB)), _full((2 * EMB, EMB)),
                      _full((8, EMB))],
            out_specs=_row_spec(BN, EMB),
            out_shape=jax.ShapeDtypeStruct((N_NODES, EMB), f32),
            compiler_params=_TC_PARAMS,
        )(nf, parts[0], parts[1], w1m_t[0:EMB], w1m_t[EMB:],
          _prep_b(b1m), _prep_w(w2m), _prep_b(b2m), _prep_w(w3m),
          _prep_b(b3m))

    pred = pl.pallas_call(
        _node_out_body,
        grid=(gn,),
        in_specs=[_row_spec(BN, EMB),
                  _full((EMB, EMB)), _full((8, EMB)), _full((EMB, EMB)),
                  _full((8, EMB)), _full((EMB, 2)), _full((8, 2))],
        out_specs=_row_spec(BN, 2),
        out_shape=jax.ShapeDtypeStruct((N_NODES, 2), f32),
        compiler_params=_TC_PARAMS,
    )(nf, _prep_w(w1o), _prep_b(b1o), _prep_w(w2o), _prep_b(b2o),
      _prep_w(w3o), _prep_b(b3o))

    return pred


# speedup vs baseline: 1.0118x; 1.0118x over previous
import functools

import jax
import jax.numpy as jnp
from jax import lax
from jax.experimental import pallas as pl
from jax.experimental.pallas import tpu as pltpu
from jax.experimental.pallas import tpu_sc as plsc

N_NODES = 50000
N_EDGES = 800000
EMB = 32
RADIUS = 0.05

NC = 2
NS = 16
NW = NC * NS
CH = 128
CPW = 196
EPW = CH * CPW
EP = NW * EPW
NCH = EP // CH
EP4 = EP // 4
DUMMY0 = N_NODES
NPAD = 50176
RPS = NPAD // NS
ZR = 196
OUT_RPS = N_NODES // NS

NCH2 = NCH // 2
CPW2 = NCH2 // NW
EP8 = EP4 // 2

_MESH = dict(core_axis_name="c", subcore_axis_name="s",
             num_cores=NC, num_subcores=NS)


def _worker_id():
    return lax.axis_index("s") * NC + lax.axis_index("c")


def _pipe(n, ig, iw, wait_i, wait_o):
    groups = (n - 4) // 3
    ig(0, 0)
    ig(1, 1)
    wait_i(0)
    iw(0, 0)
    ig(2, 2)
    wait_i(1)
    iw(1, 1)

    def grp(k, carry):
        c = 3 * k + 3
        for p in range(3):
            wait_o(p)
            ig(c + p, p)
            wait_i((p + 2) % 3)
            iw(c + p - 1, (p + 2) % 3)
        return carry

    lax.fori_loop(0, groups, grp, 0)
    for c in range(3 + 3 * groups, n):
        wait_o(c % 3)
        ig(c, c % 3)
        wait_i((c - 1) % 3)
        iw(c - 1, (c - 1) % 3)
    wait_i((n - 1) % 3)
    iw(n - 1, (n - 1) % 3)
    for b in range(3):
        wait_o(b)


@functools.cache
def _make_gather2(half: int):
    def body(tbl, idxa, idxb, outa, outb, idx_m, rows3, g0, g1, g2,
             w0, w1, w2):
        base = _worker_id() * CPW2
        sg = [g0, g1, g2]
        sw = [w0, w1, w2]

        def wait_g(b):
            pltpu.make_async_copy(outa.at[0], rows3.at[b], sg[b]).wait()

        def wait_w(b):
            pltpu.make_async_copy(outa.at[0], rows3.at[b], sw[b]).wait()

        def stream(idx_hbm, out_hbm):
            pltpu.sync_copy(idx_hbm.at[pl.ds(half * NCH2 + base, CPW2)],
                            idx_m)

            def ig(c, b):
                pltpu.async_copy(tbl.at[idx_m.at[c]], rows3.at[b], sg[b])

            def iw(c, b):
                pltpu.async_copy(rows3.at[b], out_hbm.at[base + c], sw[b])

            _pipe(CPW2, ig, iw, wait_g, wait_w)

        stream(idxa, outa)
        stream(idxb, outb)

    return pl.kernel(
        body,
        out_type=(jax.ShapeDtypeStruct((NCH2, CH, EMB), jnp.float32),
                  jax.ShapeDtypeStruct((NCH2, CH, EMB), jnp.float32)),
        mesh=plsc.VectorSubcoreMesh(**_MESH),
        scratch_types=[
            pltpu.VMEM((CPW2, CH), jnp.int32),
            pltpu.VMEM((3, CH, EMB), jnp.float32),
            pltpu.SemaphoreType.DMA,
            pltpu.SemaphoreType.DMA,
            pltpu.SemaphoreType.DMA,
            pltpu.SemaphoreType.DMA,
            pltpu.SemaphoreType.DMA,
            pltpu.SemaphoreType.DMA,
        ],
        compiler_params=pltpu.CompilerParams(use_tc_tiling_on_sc=False),
    )


@functools.cache
def _make_scatter_add():
    def body(ma_hbm, mb_hbm, dst_hbm, out_hbm, idx_r, rows3, zero_v, acc_sh,
             l0, l1, l2, a0, a1, a2):
        c = lax.axis_index("c")
        s = lax.axis_index("s")
        base = _worker_id() * CPW2
        sl = [l0, l1, l2]
        sa = [a0, a1, a2]

        def wait_l(b):
            pltpu.make_async_copy(dst_hbm.at[0], idx_r.at[b], sl[b]).wait()
            pltpu.make_async_copy(ma_hbm.at[0], rows3.at[b], sl[b]).wait()

        def wait_a(b):
            pltpu.make_async_copy(ma_hbm.at[0], rows3.at[b], sa[b]).wait()

        def stream(m_hbm, half):
            def il(ck, b):
                pltpu.async_copy(dst_hbm.at[half * NCH2 + base + ck],
                                 idx_r.at[b], sl[b])
                pltpu.async_copy(m_hbm.at[base + ck], rows3.at[b], sl[b])

            def ia(ck, b):
                del ck
                pltpu.async_copy(rows3.at[b], acc_sh.at[idx_r.at[b]], sa[b],
                                 add=True)

            _pipe(CPW2, il, ia, wait_l, wait_a)

        zv = jnp.zeros((16,), jnp.float32)

        def zloop(i, carry):
            zero_v[i, pl.ds(0, 16)] = zv
            zero_v[i, pl.ds(16, 16)] = zv
            return carry

        lax.fori_loop(0, ZR, zloop, 0)

        def zcopy(j, carry):
            pltpu.sync_copy(zero_v, acc_sh.at[pl.ds(s * RPS + j * ZR, ZR)])
            return carry

        lax.fori_loop(0, RPS // ZR, zcopy, 0)
        plsc.subcore_barrier()

        stream(ma_hbm, 0)
        stream(mb_hbm, 1)
        plsc.subcore_barrier()

        pltpu.sync_copy(acc_sh.at[pl.ds(s * RPS, RPS)],
                        out_hbm.at[c, pl.ds(s * RPS, RPS)])

    return pl.kernel(
        body,
        out_type=jax.ShapeDtypeStruct((NC, NPAD, EMB), jnp.float32),
        mesh=plsc.VectorSubcoreMesh(**_MESH),
        scratch_types=[
            pltpu.VMEM((3, CH), jnp.int32),
            pltpu.VMEM((3, CH, EMB), jnp.float32),
            pltpu.VMEM((ZR, EMB), jnp.float32),
            pltpu.VMEM_SHARED((NPAD, EMB), jnp.float32),
            pltpu.SemaphoreType.DMA,
            pltpu.SemaphoreType.DMA,
            pltpu.SemaphoreType.DMA,
            pltpu.SemaphoreType.DMA,
            pltpu.SemaphoreType.DMA,
            pltpu.SemaphoreType.DMA,
        ],
        compiler_params=pltpu.CompilerParams(use_tc_tiling_on_sc=False),
    )


BP = 2048
NP4 = NPAD // 4
BNP = NP4 // 4


_TC_PARAMS = pltpu.CompilerParams(dimension_semantics=("parallel",))


def _relu(v):
    return jnp.maximum(v, 0.0)


def _dot(a, b):
    return jnp.dot(a, b, preferred_element_type=jnp.float32)


def _full(shape):
    return pl.BlockSpec(shape, lambda i: (0, 0))


def _edge_init_body(xi_ref, xj_ref, w1_ref, b1_ref,
                    w2_ref, b2_ref, w3_ref, b3_ref, ef_ref):
    xi = xi_ref[...]
    xj = xj_ref[...]
    lane = lax.broadcasted_iota(jnp.int32, xi.shape, 1) % EMB
    dr = xi - xj
    sq = dr * dr
    r2 = sq + jnp.roll(sq, -1, axis=1)
    r = jnp.sqrt(r2) / RADIUS
    z = jnp.zeros_like(xi)
    feat = (jnp.where(lane < 2, dr / RADIUS, z)
            + jnp.where(lane == 2, jnp.roll(r, 2, axis=1), z)
            + jnp.where((lane >= 3) & (lane < 5), jnp.roll(xi, 1, axis=1), z)
            + jnp.where((lane >= 5) & (lane < 7), jnp.roll(xj, 3, axis=1), z)
            + jnp.where(lane == 7, jnp.roll(xi, 3, axis=1), z)
            + jnp.where(lane == 8, jnp.roll(xi, 4, axis=1), z)
            + jnp.where(lane == 9, jnp.roll(xi, 5, axis=1), z)
            + jnp.where(lane == 10, jnp.roll(xi, 6, axis=1), z))
    h = _relu(_dot(feat, w1_ref[...]) + b1_ref[0:1, :])
    h = _relu(_dot(h, w2_ref[...]) + b2_ref[0:1, :])
    ef_ref[...] = _dot(h, w3_ref[...]) + b3_ref[0:1, :]


def _edge_layer_body(ef_ref, xi_ref, xj_ref, w1a_ref, w1b_ref, w1c_ref,
                     b1_ref, w2_ref, b2_ref, w3_ref, b3_ref, m_ref, efo_ref):
    ef = ef_ref[...]
    h = _relu(_dot(ef, w1a_ref[...]) + _dot(xi_ref[...], w1b_ref[...])
              + _dot(xj_ref[...], w1c_ref[...]) + b1_ref[0:1, :])
    h = _relu(_dot(h, w2_ref[...]) + b2_ref[0:1, :])
    m = _dot(h, w3_ref[...]) + b3_ref[0:1, :]
    m_ref[...] = m
    efo_ref[...] = ef + m


def _edge_last_body(ef_ref, xi_ref, xj_ref, w1a_ref, w1b_ref, w1c_ref,
                    b1_ref, w2_ref, b2_ref, w3_ref, b3_ref, m_ref):
    h = _relu(_dot(ef_ref[...], w1a_ref[...]) + _dot(xi_ref[...], w1b_ref[...])
              + _dot(xj_ref[...], w1c_ref[...]) + b1_ref[0:1, :])
    h = _relu(_dot(h, w2_ref[...]) + b2_ref[0:1, :])
    m_ref[...] = _dot(h, w3_ref[...]) + b3_ref[0:1, :]


def _node_emb_body(nf_ref, w1_ref, b1_ref, w2_ref, b2_ref, w3_ref, b3_ref,
                   out_ref):
    h = _relu(_dot(nf_ref[...], w1_ref[...]) + b1_ref[0:1, :])
    h = _relu(_dot(h, w2_ref[...]) + b2_ref[0:1, :])
    out_ref[...] = _dot(h, w3_ref[...]) + b3_ref[0:1, :]


def _node_layer_body(nf_ref, p0_ref, p1_ref, w1a_ref, w1b_ref, b1_ref,
                     w2_ref, b2_ref, w3_ref, b3_ref, out_ref):
    nf = nf_ref[...]
    aggr = p0_ref[...] + p1_ref[...]
    h = _relu(_dot(nf, w1a_ref[...]) + _dot(aggr, w1b_ref[...]) + b1_ref[0:1, :])
    h = _relu(_dot(h, w2_ref[...]) + b2_ref[0:1, :])
    out_ref[...] = nf + _dot(h, w3_ref[...]) + b3_ref[0:1, :]


def _node_out_body(nf_ref, w1_ref, b1_ref, w2_ref, b2_ref, w3_ref, b3_ref,
                   out_ref):
    h = _relu(_dot(nf_ref[...], w1_ref[...]) + b1_ref[0:1, :])
    h = _relu(_dot(h, w2_ref[...]) + b2_ref[0:1, :])
    out_ref[...] = _dot(h, w3_ref[...]) + b3_ref[0:1, :]


def _row_spec(bs, w):
    return pl.BlockSpec((bs, w), lambda i: (i, 0))


def _prep_w(w):
    return jnp.asarray(w, jnp.float32).T


def _prep_b(b):
    return jnp.broadcast_to(jnp.asarray(b, jnp.float32)[None, :], (8, b.shape[0]))


def _bd4(w):
    return jax.scipy.linalg.block_diag(w, w, w, w)


def _prep_b4(b):
    return _prep_b(jnp.tile(jnp.asarray(b, jnp.float32), 4))


def kernel(x, edge_index, edge_attr, params):
    del edge_attr
    f32 = jnp.float32
    src = edge_index[0].astype(jnp.int32)
    dst = edge_index[1].astype(jnp.int32)

    nf8 = jnp.concatenate([x[:, 0:4], jnp.tile(x[:, 4:5], (1, 4))], axis=1)
    nf32 = jnp.pad(nf8.astype(f32), ((0, NPAD - N_NODES), (0, 24)))

    pad = EP - N_EDGES
    src_p = jnp.concatenate([src, jnp.zeros((pad,), jnp.int32)]).reshape(NCH, CH)
    dst_p = jnp.concatenate([dst, jnp.zeros((pad,), jnp.int32)]).reshape(NCH, CH)
    dummy = DUMMY0 + (jnp.arange(pad, dtype=jnp.int32) % (NPAD - N_NODES))
    dst_scat = jnp.concatenate([dst, dummy]).reshape(NCH, CH)

    prm = params

    (w1e, b1e), (w2e, b2e), (w3e, b3e) = prm['emb_edge']
    w1e_pad = jnp.pad(_prep_w(w1e), ((0, 21), (0, 0)))

    (w1n, b1n), (w2n, b2n), (w3n, b3n) = prm['emb_node']
    w1n_t = jnp.pad(_prep_w(w1n), ((0, 24), (0, 0)))

    (w1o, b1o), (w2o, b2o), (w3o, b3o) = prm['node_out']

    ge = EP8 // BP
    gn = NP4 // BNP

    ga = _make_gather2(0)
    gb = _make_gather2(1)

    def edge_init(xi_p, xj_p, wargs):
        return pl.pallas_call(
            _edge_init_body,
            grid=(ge,),
            in_specs=[_row_spec(BP, 128), _row_spec(BP, 128),
                      _full((128, 128)), _full((8, 128)), _full((128, 128)),
                      _full((8, 128)), _full((128, 128)), _full((8, 128))],
            out_specs=_row_spec(BP, 128),
            out_shape=jax.ShapeDtypeStruct((EP8, 128), f32),
            compiler_params=_TC_PARAMS,
        )(xi_p, xj_p, *wargs)

    def unpack(pair):
        return (jnp.reshape(pair[0], (EP8, 128)),
                jnp.reshape(pair[1], (EP8, 128)))

    xiA, xjA = unpack(ga(nf32, dst_p, src_p))
    xiB, xjB = unpack(gb(nf32, dst_p, src_p))

    ewargs = (_bd4(w1e_pad), _prep_b4(b1e), _bd4(_prep_w(w2e)),
              _prep_b4(b2e), _bd4(_prep_w(w3e)), _prep_b4(b3e))
    ef_A = edge_init(xiA, xjA, ewargs)
    ef_B = edge_init(xiB, xjB, ewargs)

    nf = pl.pallas_call(
        _node_emb_body,
        grid=(gn,),
        in_specs=[_row_spec(BNP, 128),
                  _full((128, 128)), _full((8, 128)), _full((128, 128)),
                  _full((8, 128)), _full((128, 128)), _full((8, 128))],
        out_specs=_row_spec(BNP, 128),
        out_shape=jax.ShapeDtypeStruct((NP4, 128), f32),
        compiler_params=_TC_PARAMS,
    )(jnp.reshape(nf32, (NP4, 128)), _bd4(w1n_t), _prep_b4(b1n),
      _bd4(_prep_w(w2n)), _prep_b4(b2n), _bd4(_prep_w(w3n)), _prep_b4(b3n))

    n_layers = len(prm['layers'])
    for li, lp in enumerate(prm['layers']):
        (w1l, b1l), (w2l, b2l), (w3l, b3l) = lp['lin_edge']
        w1l_t = _prep_w(w1l)
        (w1m, b1m), (w2m, b2m), (w3m, b3m) = lp['lin_node']
        w1m_t = _prep_w(w1m)

        last = li == n_layers - 1
        wargs = (_bd4(w1l_t[0:EMB]), _bd4(w1l_t[EMB:2 * EMB]),
                 _bd4(w1l_t[2 * EMB:]), _prep_b4(b1l), _bd4(_prep_w(w2l)),
                 _prep_b4(b2l), _bd4(_prep_w(w3l)), _prep_b4(b3l))
        especs = [_row_spec(BP, 128), _row_spec(BP, 128), _row_spec(BP, 128),
                  _full((128, 384)), _full((128, 384)),
                  _full((128, 384)), _full((8, 384)),
                  _full((384, 384)), _full((8, 384)),
                  _full((384, 128)), _full((8, 128))]

        def edge_layer(ef_h, xi_p, xj_p):
            if last:
                m = pl.pallas_call(
                    _edge_last_body,
                    grid=(ge,),
                    in_specs=especs,
                    out_specs=_row_spec(BP, 128),
                    out_shape=jax.ShapeDtypeStruct((EP8, 128), f32),
                    compiler_params=_TC_PARAMS,
                )(ef_h, xi_p, xj_p, *wargs)
                return m, m
            return pl.pallas_call(
                _edge_layer_body,
                grid=(ge,),
                in_specs=especs,
                out_specs=(_row_spec(BP, 128), _row_spec(BP, 128)),
                out_shape=(jax.ShapeDtypeStruct((EP8, 128), f32),
                           jax.ShapeDtypeStruct((EP8, 128), f32)),
                compiler_params=_TC_PARAMS,
            )(ef_h, xi_p, xj_p, *wargs)

        nf_tbl = jnp.reshape(nf, (NPAD, EMB))
        xiA, xjA = unpack(ga(nf_tbl, dst_p, src_p))
        xiB, xjB = unpack(gb(nf_tbl, dst_p, src_p))
        m_A, ef_A = edge_layer(ef_A, xiA, xjA)
        m_B, ef_B = edge_layer(ef_B, xiB, xjB)

        parts = _make_scatter_add()(jnp.reshape(m_A, (NCH2, CH, EMB)),
                                    jnp.reshape(m_B, (NCH2, CH, EMB)),
                                    dst_scat)

        nf = pl.pallas_call(
            _node_layer_body,
            grid=(gn,),
            in_specs=[_row_spec(BNP, 128), _row_spec(BNP, 128),
                      _row_spec(BNP, 128),
                      _full((128, 256)), _full((128, 256)),
                      _full((8, 256)), _full((256, 256)),
                      _full((8, 256)), _full((256, 128)),
                      _full((8, 128))],
            out_specs=_row_spec(BNP, 128),
            out_shape=jax.ShapeDtypeStruct((NP4, 128), f32),
            compiler_params=_TC_PARAMS,
        )(nf, jnp.reshape(parts[0], (NP4, 128)),
          jnp.reshape(parts[1], (NP4, 128)),
          _bd4(w1m_t[0:EMB]), _bd4(w1m_t[EMB:]),
          _prep_b4(b1m), _bd4(_prep_w(w2m)), _prep_b4(b2m),
          _bd4(_prep_w(w3m)), _prep_b4(b3m))

    pred = pl.pallas_call(
        _node_out_body,
        grid=(gn,),
        in_specs=[_row_spec(BNP, 128),
                  _full((128, 128)), _full((8, 128)), _full((128, 128)),
                  _full((8, 128)), _full((128, 8)), _full((8, 8))],
        out_specs=_row_spec(BNP, 8),
        out_shape=jax.ShapeDtypeStruct((NP4, 8), f32),
        compiler_params=_TC_PARAMS,
    )(nf, _bd4(_prep_w(w1o)), _prep_b4(b1o), _bd4(_prep_w(w2o)),
      _prep_b4(b2o), _bd4(_prep_w(w3o)), _prep_b4(b3o))

    return jnp.reshape(pred, (NPAD, 2))[:N_NODES]
